# Initial kernel scaffold; baseline (speedup 1.0000x reference)
import jax
import jax.numpy as jnp
from jax.experimental import pallas as pl


def kernel(features, labels, is_train_mask, W_feat, b_feat, feat_idx_table, label_table, is_train_table, pos_table):
    raise NotImplementedError("write your pallas kernel here")


# speedup vs baseline: 32.2187x; 32.2187x over previous
import functools

import jax
import jax.numpy as jnp
from jax.experimental import pallas as pl


_B, _S, _F, _D = 2, 2048, 64, 256
_MAX_CLASSES = 10
_S_CHUNK = 64
_NSB = _S // _S_CHUNK


def _encoder_block(feats_ref, labels_ref, mask_ref, w_ref, bias_ref,
                   feat_tab_ref, label_tab_ref, train_tab_ref, pos_ref,
                   out_ref):
    lab = labels_ref[0, 0, :]
    m = mask_ref[0, 0, :]
    lab_eff = lab * m + _MAX_CLASSES * (1 - m)

    classes = jax.lax.broadcasted_iota(jnp.int32, (_S_CHUNK, _MAX_CLASSES + 1), 1)
    onehot = (lab_eff[:, None] == classes).astype(jnp.float32)
    lab_emb = jnp.dot(onehot, label_tab_ref[...],
                      preferred_element_type=jnp.float32)

    t0 = train_tab_ref[0, :]
    t1 = train_tab_ref[1, :]
    m_f = m.astype(jnp.float32)[:, None]
    train_emb = t0[None, :] + m_f * (t1 - t0)[None, :]

    row = pos_ref[...] + lab_emb + train_emb

    base_f = bias_ref[...] + feat_tab_ref[...]

    feats = feats_ref[0]
    w = w_ref[0, :]
    full = (feats[:, :, None] * w[None, None, :]
            + base_f[None, :, :] + row[:, None, :])
    out_ref[...] = full.reshape(1, _S_CHUNK * _F, _D)


@jax.jit
def kernel(features, labels, is_train_mask, W_feat, b_feat, feat_idx_table,
           label_table, is_train_table, pos_table):
    b, s, f = features.shape
    d = W_feat.shape[1]
    labels = labels.astype(jnp.int32)
    is_train_mask = is_train_mask.astype(jnp.int32)

    grid = (_B, _NSB)
    out = pl.pallas_call(
        _encoder_block,
        grid=grid,
        in_specs=[
            pl.BlockSpec((1, _S_CHUNK, _F), lambda b, sb: (b, sb, 0)),
            pl.BlockSpec((1, 1, _S_CHUNK), lambda b, sb: (b * _NSB + sb, 0, 0)),
            pl.BlockSpec((1, 1, _S_CHUNK), lambda b, sb: (b * _NSB + sb, 0, 0)),
            pl.BlockSpec((1, _D), lambda b, sb: (0, 0)),
            pl.BlockSpec((1, _D), lambda b, sb: (0, 0)),
            pl.BlockSpec((_F, _D), lambda b, sb: (0, 0)),
            pl.BlockSpec((_MAX_CLASSES + 1, _D), lambda b, sb: (0, 0)),
            pl.BlockSpec((2, _D), lambda b, sb: (0, 0)),
            pl.BlockSpec((_S_CHUNK, _D), lambda b, sb: (sb, 0)),
        ],
        out_specs=pl.BlockSpec((1, _S_CHUNK * _F, _D), lambda b, sb: (b, sb, 0)),
        out_shape=jax.ShapeDtypeStruct((b, s * f, d), jnp.float32),
    )(features, labels.reshape(_B * _NSB, 1, _S_CHUNK),
      is_train_mask.reshape(_B * _NSB, 1, _S_CHUNK), W_feat, b_feat.reshape(1, d),
      feat_idx_table, label_table, is_train_table, pos_table)
    return out


# speedup vs baseline: 36.0683x; 1.1195x over previous
import functools

import jax
import jax.numpy as jnp
from jax.experimental import pallas as pl


_B, _S, _F, _D = 2, 2048, 64, 256
_MAX_CLASSES = 10
_S_CHUNK = 128
_NSB = _S // _S_CHUNK


def _encoder_block(feats_ref, labels_ref, mask_ref, w_ref, bias_ref,
                   feat_tab_ref, label_tab_ref, train_tab_ref, pos_ref,
                   out_ref):
    lab = labels_ref[0, 0, :]
    m = mask_ref[0, 0, :]
    lab_eff = lab * m + _MAX_CLASSES * (1 - m)

    classes = jax.lax.broadcasted_iota(jnp.int32, (_S_CHUNK, _MAX_CLASSES + 1), 1)
    onehot = (lab_eff[:, None] == classes).astype(jnp.float32)
    lab_emb = jnp.dot(onehot, label_tab_ref[...],
                      preferred_element_type=jnp.float32)

    t0 = train_tab_ref[0, :]
    t1 = train_tab_ref[1, :]
    m_f = m.astype(jnp.float32)[:, None]
    train_emb = t0[None, :] + m_f * (t1 - t0)[None, :]

    row = pos_ref[...] + lab_emb + train_emb

    base_f = bias_ref[...] + feat_tab_ref[...]

    feats = feats_ref[0]
    w = w_ref[0, :]
    full = (feats[:, :, None] * w[None, None, :]
            + base_f[None, :, :] + row[:, None, :])
    out_ref[...] = full.reshape(1, _S_CHUNK * _F, _D)


@jax.jit
def kernel(features, labels, is_train_mask, W_feat, b_feat, feat_idx_table,
           label_table, is_train_table, pos_table):
    b, s, f = features.shape
    d = W_feat.shape[1]
    labels = labels.astype(jnp.int32)
    is_train_mask = is_train_mask.astype(jnp.int32)

    grid = (_B, _NSB)
    out = pl.pallas_call(
        _encoder_block,
        grid=grid,
        in_specs=[
            pl.BlockSpec((1, _S_CHUNK, _F), lambda b, sb: (b, sb, 0)),
            pl.BlockSpec((1, 1, _S_CHUNK), lambda b, sb: (b * _NSB + sb, 0, 0)),
            pl.BlockSpec((1, 1, _S_CHUNK), lambda b, sb: (b * _NSB + sb, 0, 0)),
            pl.BlockSpec((1, _D), lambda b, sb: (0, 0)),
            pl.BlockSpec((1, _D), lambda b, sb: (0, 0)),
            pl.BlockSpec((_F, _D), lambda b, sb: (0, 0)),
            pl.BlockSpec((_MAX_CLASSES + 1, _D), lambda b, sb: (0, 0)),
            pl.BlockSpec((2, _D), lambda b, sb: (0, 0)),
            pl.BlockSpec((_S_CHUNK, _D), lambda b, sb: (sb, 0)),
        ],
        out_specs=pl.BlockSpec((1, _S_CHUNK * _F, _D), lambda b, sb: (b, sb, 0)),
        out_shape=jax.ShapeDtypeStruct((b, s * f, d), jnp.float32),
    )(features, labels.reshape(_B * _NSB, 1, _S_CHUNK),
      is_train_mask.reshape(_B * _NSB, 1, _S_CHUNK), W_feat, b_feat.reshape(1, d),
      feat_idx_table, label_table, is_train_table, pos_table)
    return out
